# baseline (speedup 1.0000x reference)
import functools

import jax
import jax.numpy as jnp
from jax import lax
from jax.experimental import pallas as pl
from jax.experimental.pallas import tpu as pltpu
from jax.experimental.pallas import tpu_sc as plsc

N = 10000
E = 320000
D = 128

NTILES = 32
R = 10240
RPT = R // 16
CHUNK = 128
EPW = 10240
NCHUNK = EPW // CHUNK
EPAD = NTILES * EPW
DUMMY = N

_mesh = plsc.VectorSubcoreMesh(core_axis_name="c", subcore_axis_name="s")
_prec = jax.lax.Precision.HIGHEST


@functools.partial(
    pl.kernel,
    out_type=jax.ShapeDtypeStruct((2, R), jnp.float32),
    mesh=_mesh,
    scratch_types=[
        pltpu.VMEM((NCHUNK // 5, CHUNK), jnp.int32),
        pltpu.VMEM((CHUNK,), jnp.float32),
        pltpu.VMEM((RPT,), jnp.float32),
        pltpu.SemaphoreType.DMA,
        pltpu.VMEM_SHARED((R,), jnp.float32),
    ],
)
def _deg_kernel(dst_hbm, out_hbm, dst_v, ones_v, zeros_v, sem, hist_sh):
    c = lax.axis_index("c")
    s = lax.axis_index("s")
    w = c * 16 + s

    def _z16(k, carry):
        zeros_v[pl.ds(k * 16, 16)] = jnp.zeros((16,), jnp.float32)
        return carry

    lax.fori_loop(0, RPT // 16, _z16, 0)

    def _o16(k, carry):
        ones_v[pl.ds(k * 16, 16)] = jnp.ones((16,), jnp.float32)
        return carry

    lax.fori_loop(0, CHUNK // 16, _o16, 0)

    pltpu.sync_copy(zeros_v, hist_sh.at[pl.ds(s * RPT, RPT)])
    plsc.subcore_barrier()

    fifth = NCHUNK // 5
    for q in range(5):
        pltpu.sync_copy(dst_hbm.at[pl.ds(w * NCHUNK + q * fifth, fifth)], dst_v)

        def _fire8(g, carry):
            for b in range(8):
                pltpu.make_async_copy(
                    ones_v, hist_sh.at[dst_v.at[g * 8 + b]], sem
                ).start(add=True)
            for b in range(8):
                pltpu.make_async_copy(
                    ones_v, hist_sh.at[dst_v.at[g * 8 + b]], sem
                ).wait()
            return carry

        lax.fori_loop(0, fifth // 8, _fire8, 0)
    plsc.subcore_barrier()
    pltpu.sync_copy(
        hist_sh.at[pl.ds(s * RPT, RPT)], out_hbm.at[c, pl.ds(s * RPT, RPT)]
    )


@functools.partial(
    pl.kernel,
    out_type=jax.ShapeDtypeStruct((2, R, D), jnp.float32),
    mesh=_mesh,
    scratch_types=[
        pltpu.VMEM((NCHUNK // 2, CHUNK), jnp.int32),
        pltpu.VMEM((NCHUNK // 2, CHUNK), jnp.int32),
        pltpu.VMEM((2, CHUNK, D), jnp.float32),
        pltpu.SemaphoreType.DMA,
        pltpu.SemaphoreType.DMA,
        pltpu.SemaphoreType.DMA,
        pltpu.SemaphoreType.DMA,
        pltpu.VMEM_SHARED((R, D), jnp.float32),
    ],
)
def _agg_kernel(y_hbm, src_hbm, dst_hbm, out_hbm, src_v, dst_v, rows_v,
                gsem0, gsem1, ssem0, ssem1, acc_sh):
    c = lax.axis_index("c")
    s = lax.axis_index("s")
    w = c * 16 + s
    pltpu.sync_copy(y_hbm.at[pl.ds(s * RPT, RPT)], acc_sh.at[pl.ds(s * RPT, RPT)])
    plsc.subcore_barrier()

    gsems = (gsem0, gsem1)
    ssems = (ssem0, ssem1)
    half = NCHUNK // 2
    for h in range(2):
        base = w * NCHUNK + h * half
        pltpu.sync_copy(src_hbm.at[pl.ds(base, half)], src_v)
        pltpu.sync_copy(dst_hbm.at[pl.ds(base, half)], dst_v)
        pltpu.make_async_copy(y_hbm.at[src_v.at[0]], rows_v.at[0], gsem0).start()

        def _body(t, carry):
            for b in range(2):
                j = t * 2 + b
                nxt = j + 1

                @pl.when(j >= 1)
                def _():
                    pltpu.make_async_copy(
                        rows_v.at[1 - b], acc_sh.at[dst_v.at[0]], ssems[1 - b]
                    ).wait()

                @pl.when(nxt < half)
                def _():
                    pltpu.make_async_copy(
                        y_hbm.at[src_v.at[nxt]], rows_v.at[1 - b], gsems[1 - b]
                    ).start()

                pltpu.make_async_copy(
                    y_hbm.at[src_v.at[j]], rows_v.at[b], gsems[b]
                ).wait()
                pltpu.make_async_copy(
                    rows_v.at[b], acc_sh.at[dst_v.at[j]], ssems[b]
                ).start(add=True)
            return carry

        lax.fori_loop(0, half // 2, _body, 0)
        pltpu.make_async_copy(rows_v.at[1], acc_sh.at[dst_v.at[0]], ssem1).wait()
    plsc.subcore_barrier()
    pltpu.sync_copy(
        acc_sh.at[pl.ds(s * RPT, RPT)], out_hbm.at[c, pl.ds(s * RPT, RPT)]
    )


BLK = 400
NBLK = N // BLK


def _mm_body(x_ref, w_ref, xw_ref):
    xw_ref[...] = jnp.dot(x_ref[...], w_ref[...],
                          preferred_element_type=jnp.float32, precision=_prec)


def _mm(x, W1):
    return pl.pallas_call(
        _mm_body,
        grid=(NBLK,),
        in_specs=[
            pl.BlockSpec((BLK, D), lambda i: (i, 0)),
            pl.BlockSpec((D, D), lambda i: (0, 0)),
        ],
        out_specs=pl.BlockSpec((BLK, D), lambda i: (i, 0)),
        out_shape=jax.ShapeDtypeStruct((N, D), jnp.float32),
    )(x, W1)


def _scale_body(degp_ref, xw_ref, y_ref, dinv_ref):
    deg = degp_ref[0] + degp_ref[1] + 1.0
    dinv = lax.rsqrt(deg)
    y_ref[...] = xw_ref[...] * dinv
    dinv_ref[...] = dinv


def _scale(degp, xw):
    return pl.pallas_call(
        _scale_body,
        grid=(NBLK,),
        in_specs=[
            pl.BlockSpec((2, BLK, 1), lambda i: (0, i, 0)),
            pl.BlockSpec((BLK, D), lambda i: (i, 0)),
        ],
        out_specs=[
            pl.BlockSpec((BLK, D), lambda i: (i, 0)),
            pl.BlockSpec((BLK, 1), lambda i: (i, 0)),
        ],
        out_shape=[
            jax.ShapeDtypeStruct((R, D), jnp.float32),
            jax.ShapeDtypeStruct((R, 1), jnp.float32),
        ],
    )(degp, xw)


def _mid_body(p_ref, y1_ref, dinv_ref, b1_ref, w2_ref, y2_ref):
    agg = p_ref[0] + p_ref[1] - y1_ref[...]
    h = jnp.maximum(agg * dinv_ref[...] + b1_ref[...], 0.0)
    y2_ref[...] = jnp.dot(h, w2_ref[...], preferred_element_type=jnp.float32,
                          precision=_prec) * dinv_ref[...]


def _mid(p, y1, dinv, b1, W2):
    return pl.pallas_call(
        _mid_body,
        grid=(NBLK,),
        in_specs=[
            pl.BlockSpec((2, BLK, D), lambda i: (0, i, 0)),
            pl.BlockSpec((BLK, D), lambda i: (i, 0)),
            pl.BlockSpec((BLK, 1), lambda i: (i, 0)),
            pl.BlockSpec((1, D), lambda i: (0, 0)),
            pl.BlockSpec((D, D), lambda i: (0, 0)),
        ],
        out_specs=pl.BlockSpec((BLK, D), lambda i: (i, 0)),
        out_shape=jax.ShapeDtypeStruct((R, D), jnp.float32),
    )(p, y1, dinv, b1, W2)


def _final_body(q_ref, y2_ref, dinv_ref, b2_ref, out_ref):
    agg = q_ref[0] + q_ref[1] - y2_ref[...]
    out_ref[...] = agg * dinv_ref[...] + b2_ref[...]


def _final(q, y2, dinv, b2):
    return pl.pallas_call(
        _final_body,
        grid=(NBLK,),
        in_specs=[
            pl.BlockSpec((2, BLK, D), lambda i: (0, i, 0)),
            pl.BlockSpec((BLK, D), lambda i: (i, 0)),
            pl.BlockSpec((BLK, 1), lambda i: (i, 0)),
            pl.BlockSpec((1, D), lambda i: (0, 0)),
        ],
        out_specs=pl.BlockSpec((BLK, D), lambda i: (i, 0)),
        out_shape=jax.ShapeDtypeStruct((N, D), jnp.float32),
    )(q, y2, dinv, b2)


def kernel(x, edge_index, W1, b1, W2, b2):
    pad = DUMMY + jnp.arange(EPAD - E, dtype=jnp.int32) % (R - N)
    src2d = jnp.reshape(jnp.concatenate([edge_index[0], pad]), (EPAD // CHUNK, CHUNK))
    dst2d = jnp.reshape(jnp.concatenate([edge_index[1], pad]), (EPAD // CHUNK, CHUNK))

    degp = _deg_kernel(dst2d).reshape(2, R, 1)
    xw1 = _mm(x, W1)
    y1, dinv = _scale(degp, xw1)
    p = _agg_kernel(y1, src2d, dst2d)
    y2 = _mid(p, y1, dinv, b1.reshape(1, D), W2)
    q = _agg_kernel(y2, src2d, dst2d)
    return _final(q, y2, dinv, b2.reshape(1, D))


# speedup vs baseline: 1.0230x; 1.0230x over previous
import functools

import jax
import jax.numpy as jnp
from jax import lax
from jax.experimental import pallas as pl
from jax.experimental.pallas import tpu as pltpu
from jax.experimental.pallas import tpu_sc as plsc

N = 10000
E = 320000
D = 128

NTILES = 32
R = 10240
RPT = R // 16
CHUNK = 128
EPW = 10240
NCHUNK = EPW // CHUNK
EPAD = NTILES * EPW
DUMMY = N

_mesh = plsc.VectorSubcoreMesh(core_axis_name="c", subcore_axis_name="s")
_prec = jax.lax.Precision.HIGHEST


@functools.partial(
    pl.kernel,
    out_type=jax.ShapeDtypeStruct((2, R), jnp.float32),
    mesh=_mesh,
    scratch_types=[
        pltpu.VMEM((NCHUNK // 5, CHUNK), jnp.int32),
        pltpu.VMEM((CHUNK,), jnp.float32),
        pltpu.VMEM((RPT,), jnp.float32),
        pltpu.SemaphoreType.DMA,
        pltpu.VMEM_SHARED((R,), jnp.float32),
    ],
)
def _deg_kernel(dst_hbm, out_hbm, dst_v, ones_v, zeros_v, sem, hist_sh):
    c = lax.axis_index("c")
    s = lax.axis_index("s")
    w = c * 16 + s

    def _z16(k, carry):
        zeros_v[pl.ds(k * 16, 16)] = jnp.zeros((16,), jnp.float32)
        return carry

    lax.fori_loop(0, RPT // 16, _z16, 0)

    def _o16(k, carry):
        ones_v[pl.ds(k * 16, 16)] = jnp.ones((16,), jnp.float32)
        return carry

    lax.fori_loop(0, CHUNK // 16, _o16, 0)

    pltpu.sync_copy(zeros_v, hist_sh.at[pl.ds(s * RPT, RPT)])
    plsc.subcore_barrier()

    fifth = NCHUNK // 5
    for q in range(5):
        pltpu.sync_copy(dst_hbm.at[pl.ds(w * NCHUNK + q * fifth, fifth)], dst_v)

        def _fire8(g, carry):
            for b in range(8):
                pltpu.make_async_copy(
                    ones_v, hist_sh.at[dst_v.at[g * 8 + b]], sem
                ).start(add=True)
            for b in range(8):
                pltpu.make_async_copy(
                    ones_v, hist_sh.at[dst_v.at[g * 8 + b]], sem
                ).wait()
            return carry

        lax.fori_loop(0, fifth // 8, _fire8, 0)
    plsc.subcore_barrier()
    pltpu.sync_copy(
        hist_sh.at[pl.ds(s * RPT, RPT)], out_hbm.at[c, pl.ds(s * RPT, RPT)]
    )


@functools.partial(
    pl.kernel,
    out_type=jax.ShapeDtypeStruct((2, R, D), jnp.float32),
    mesh=_mesh,
    scratch_types=[
        pltpu.VMEM((NCHUNK // 2, CHUNK), jnp.int32),
        pltpu.VMEM((NCHUNK // 2, CHUNK), jnp.int32),
        pltpu.VMEM((2, CHUNK, D), jnp.float32),
        pltpu.SemaphoreType.DMA,
        pltpu.SemaphoreType.DMA,
        pltpu.SemaphoreType.DMA,
        pltpu.SemaphoreType.DMA,
        pltpu.VMEM_SHARED((R, D), jnp.float32),
    ],
)
def _agg_kernel(y_hbm, src_hbm, dst_hbm, out_hbm, src_v, dst_v, rows_v,
                gsem0, gsem1, ssem0, ssem1, acc_sh):
    c = lax.axis_index("c")
    s = lax.axis_index("s")
    w = c * 16 + s
    pltpu.sync_copy(y_hbm.at[pl.ds(s * RPT, RPT)], acc_sh.at[pl.ds(s * RPT, RPT)])
    plsc.subcore_barrier()

    gsems = (gsem0, gsem1)
    ssems = (ssem0, ssem1)
    half = NCHUNK // 2
    for h in range(2):
        base = w * NCHUNK + h * half
        pltpu.sync_copy(src_hbm.at[pl.ds(base, half)], src_v)
        pltpu.sync_copy(dst_hbm.at[pl.ds(base, half)], dst_v)
        pltpu.make_async_copy(y_hbm.at[src_v.at[0]], rows_v.at[0], gsem0).start()

        def _body(t, carry):
            for b in range(2):
                j = t * 2 + b
                nxt = j + 1

                @pl.when(j >= 1)
                def _():
                    pltpu.make_async_copy(
                        rows_v.at[1 - b], acc_sh.at[dst_v.at[0]], ssems[1 - b]
                    ).wait()

                @pl.when(nxt < half)
                def _():
                    pltpu.make_async_copy(
                        y_hbm.at[src_v.at[nxt]], rows_v.at[1 - b], gsems[1 - b]
                    ).start()

                pltpu.make_async_copy(
                    y_hbm.at[src_v.at[j]], rows_v.at[b], gsems[b]
                ).wait()
                pltpu.make_async_copy(
                    rows_v.at[b], acc_sh.at[dst_v.at[j]], ssems[b]
                ).start(add=True)
            return carry

        lax.fori_loop(0, half // 2, _body, 0)
        pltpu.make_async_copy(rows_v.at[1], acc_sh.at[dst_v.at[0]], ssem1).wait()
    plsc.subcore_barrier()
    pltpu.sync_copy(
        acc_sh.at[pl.ds(s * RPT, RPT)], out_hbm.at[c, pl.ds(s * RPT, RPT)]
    )


BLK = 400
NBLK = N // BLK


def _scale_mm_body(degp_ref, x_ref, w_ref, y_ref, dinv_ref):
    deg = degp_ref[0] + degp_ref[1] + 1.0
    dinv = lax.rsqrt(deg)
    xw = jnp.dot(x_ref[...], w_ref[...], preferred_element_type=jnp.float32,
                 precision=_prec)
    y_ref[...] = xw * dinv
    dinv_ref[...] = dinv


def _scale_matmul(degp, x, W1):
    return pl.pallas_call(
        _scale_mm_body,
        grid=(NBLK,),
        in_specs=[
            pl.BlockSpec((2, BLK, 1), lambda i: (0, i, 0)),
            pl.BlockSpec((BLK, D), lambda i: (i, 0)),
            pl.BlockSpec((D, D), lambda i: (0, 0)),
        ],
        out_specs=[
            pl.BlockSpec((BLK, D), lambda i: (i, 0)),
            pl.BlockSpec((BLK, 1), lambda i: (i, 0)),
        ],
        out_shape=[
            jax.ShapeDtypeStruct((R, D), jnp.float32),
            jax.ShapeDtypeStruct((R, 1), jnp.float32),
        ],
    )(degp, x, W1)


def _mid_body(p_ref, y1_ref, dinv_ref, b1_ref, w2_ref, y2_ref):
    agg = p_ref[0] + p_ref[1] - y1_ref[...]
    h = jnp.maximum(agg * dinv_ref[...] + b1_ref[...], 0.0)
    y2_ref[...] = jnp.dot(h, w2_ref[...], preferred_element_type=jnp.float32,
                          precision=_prec) * dinv_ref[...]


def _mid(p, y1, dinv, b1, W2):
    return pl.pallas_call(
        _mid_body,
        grid=(NBLK,),
        in_specs=[
            pl.BlockSpec((2, BLK, D), lambda i: (0, i, 0)),
            pl.BlockSpec((BLK, D), lambda i: (i, 0)),
            pl.BlockSpec((BLK, 1), lambda i: (i, 0)),
            pl.BlockSpec((1, D), lambda i: (0, 0)),
            pl.BlockSpec((D, D), lambda i: (0, 0)),
        ],
        out_specs=pl.BlockSpec((BLK, D), lambda i: (i, 0)),
        out_shape=jax.ShapeDtypeStruct((R, D), jnp.float32),
    )(p, y1, dinv, b1, W2)


def _final_body(q_ref, y2_ref, dinv_ref, b2_ref, out_ref):
    agg = q_ref[0] + q_ref[1] - y2_ref[...]
    out_ref[...] = agg * dinv_ref[...] + b2_ref[...]


def _final(q, y2, dinv, b2):
    return pl.pallas_call(
        _final_body,
        grid=(NBLK,),
        in_specs=[
            pl.BlockSpec((2, BLK, D), lambda i: (0, i, 0)),
            pl.BlockSpec((BLK, D), lambda i: (i, 0)),
            pl.BlockSpec((BLK, 1), lambda i: (i, 0)),
            pl.BlockSpec((1, D), lambda i: (0, 0)),
        ],
        out_specs=pl.BlockSpec((BLK, D), lambda i: (i, 0)),
        out_shape=jax.ShapeDtypeStruct((N, D), jnp.float32),
    )(q, y2, dinv, b2)


def kernel(x, edge_index, W1, b1, W2, b2):
    pad = DUMMY + jnp.arange(EPAD - E, dtype=jnp.int32) % (R - N)
    src2d = jnp.reshape(jnp.concatenate([edge_index[0], pad]), (EPAD // CHUNK, CHUNK))
    dst2d = jnp.reshape(jnp.concatenate([edge_index[1], pad]), (EPAD // CHUNK, CHUNK))

    degp = _deg_kernel(dst2d).reshape(2, R, 1)
    y1, dinv = _scale_matmul(degp, x, W1)
    p = _agg_kernel(y1, src2d, dst2d)
    y2 = _mid(p, y1, dinv, b1.reshape(1, D), W2)
    q = _agg_kernel(y2, src2d, dst2d)
    return _final(q, y2, dinv, b2.reshape(1, D))


# speedup vs baseline: 1.0294x; 1.0063x over previous
import functools

import jax
import jax.numpy as jnp
import numpy as np
from jax import lax
from jax.experimental import pallas as pl
from jax.experimental.pallas import tpu as pltpu
from jax.experimental.pallas import tpu_sc as plsc

N = 10000
E = 320000
D = 128

NTILES = 32
R = 10240
RPT = R // 16
CHUNK = 128
EPW = 10240
NCHUNK = EPW // CHUNK
EPAD = NTILES * EPW
DUMMY = N

_mesh = plsc.VectorSubcoreMesh(core_axis_name="c", subcore_axis_name="s")
_prec = None

_PAD = np.asarray(N + np.arange(EPAD - E) % (R - N), np.int32)


@functools.partial(
    pl.kernel,
    out_type=jax.ShapeDtypeStruct((2, R), jnp.float32),
    mesh=_mesh,
    scratch_types=[
        pltpu.VMEM((NCHUNK // 5, CHUNK), jnp.int32),
        pltpu.VMEM((CHUNK,), jnp.float32),
        pltpu.VMEM((RPT,), jnp.float32),
        pltpu.SemaphoreType.DMA,
        pltpu.VMEM_SHARED((R,), jnp.float32),
    ],
)
def _deg_kernel(dst_hbm, out_hbm, dst_v, ones_v, zeros_v, sem, hist_sh):
    c = lax.axis_index("c")
    s = lax.axis_index("s")
    w = c * 16 + s

    def _z16(k, carry):
        zeros_v[pl.ds(k * 16, 16)] = jnp.zeros((16,), jnp.float32)
        return carry

    lax.fori_loop(0, RPT // 16, _z16, 0)

    def _o16(k, carry):
        ones_v[pl.ds(k * 16, 16)] = jnp.ones((16,), jnp.float32)
        return carry

    lax.fori_loop(0, CHUNK // 16, _o16, 0)

    pltpu.sync_copy(zeros_v, hist_sh.at[pl.ds(s * RPT, RPT)])
    plsc.subcore_barrier()

    fifth = NCHUNK // 5
    for q in range(5):
        pltpu.sync_copy(dst_hbm.at[pl.ds(w * NCHUNK + q * fifth, fifth)], dst_v)

        def _fire8(g, carry):
            for b in range(8):
                pltpu.make_async_copy(
                    ones_v, hist_sh.at[dst_v.at[g * 8 + b]], sem
                ).start(add=True)
            for b in range(8):
                pltpu.make_async_copy(
                    ones_v, hist_sh.at[dst_v.at[g * 8 + b]], sem
                ).wait()
            return carry

        lax.fori_loop(0, fifth // 8, _fire8, 0)
    plsc.subcore_barrier()
    pltpu.sync_copy(
        hist_sh.at[pl.ds(s * RPT, RPT)], out_hbm.at[c, pl.ds(s * RPT, RPT)]
    )


@functools.partial(
    pl.kernel,
    out_type=jax.ShapeDtypeStruct((2, R, D), jnp.float32),
    mesh=_mesh,
    scratch_types=[
        pltpu.VMEM((NCHUNK // 2, CHUNK), jnp.int32),
        pltpu.VMEM((NCHUNK // 2, CHUNK), jnp.int32),
        pltpu.VMEM((2, CHUNK, D), jnp.float32),
        pltpu.SemaphoreType.DMA,
        pltpu.SemaphoreType.DMA,
        pltpu.SemaphoreType.DMA,
        pltpu.SemaphoreType.DMA,
        pltpu.VMEM_SHARED((R, D), jnp.float32),
    ],
)
def _agg_kernel(y_hbm, src_hbm, dst_hbm, out_hbm, src_v, dst_v, rows_v,
                gsem0, gsem1, ssem0, ssem1, acc_sh):
    c = lax.axis_index("c")
    s = lax.axis_index("s")
    w = c * 16 + s
    pltpu.sync_copy(y_hbm.at[pl.ds(s * RPT, RPT)], acc_sh.at[pl.ds(s * RPT, RPT)])
    plsc.subcore_barrier()

    gsems = (gsem0, gsem1)
    ssems = (ssem0, ssem1)
    half = NCHUNK // 2
    for h in range(2):
        base = w * NCHUNK + h * half
        pltpu.sync_copy(src_hbm.at[pl.ds(base, half)], src_v)
        pltpu.sync_copy(dst_hbm.at[pl.ds(base, half)], dst_v)
        pltpu.make_async_copy(y_hbm.at[src_v.at[0]], rows_v.at[0], gsem0).start()

        def _body(t, carry):
            for b in range(2):
                j = t * 2 + b
                nxt = j + 1

                @pl.when(j >= 1)
                def _():
                    pltpu.make_async_copy(
                        rows_v.at[1 - b], acc_sh.at[dst_v.at[0]], ssems[1 - b]
                    ).wait()

                @pl.when(nxt < half)
                def _():
                    pltpu.make_async_copy(
                        y_hbm.at[src_v.at[nxt]], rows_v.at[1 - b], gsems[1 - b]
                    ).start()

                pltpu.make_async_copy(
                    y_hbm.at[src_v.at[j]], rows_v.at[b], gsems[b]
                ).wait()
                pltpu.make_async_copy(
                    rows_v.at[b], acc_sh.at[dst_v.at[j]], ssems[b]
                ).start(add=True)
            return carry

        lax.fori_loop(0, half // 2, _body, 0)
        pltpu.make_async_copy(rows_v.at[1], acc_sh.at[dst_v.at[0]], ssem1).wait()
    plsc.subcore_barrier()
    pltpu.sync_copy(
        acc_sh.at[pl.ds(s * RPT, RPT)], out_hbm.at[c, pl.ds(s * RPT, RPT)]
    )


BLK = 400
NBLK = N // BLK


def _scale_mm_body(degp_ref, x_ref, w_ref, y_ref, dinv_ref):
    deg = degp_ref[0] + degp_ref[1] + 1.0
    dinv = lax.rsqrt(deg)
    xw = jnp.dot(x_ref[...], w_ref[...], preferred_element_type=jnp.float32,
                 precision=_prec)
    y_ref[...] = xw * dinv
    dinv_ref[...] = dinv


def _scale_matmul(degp, x, W1):
    return pl.pallas_call(
        _scale_mm_body,
        grid=(NBLK,),
        in_specs=[
            pl.BlockSpec((2, BLK, 1), lambda i: (0, i, 0)),
            pl.BlockSpec((BLK, D), lambda i: (i, 0)),
            pl.BlockSpec((D, D), lambda i: (0, 0)),
        ],
        out_specs=[
            pl.BlockSpec((BLK, D), lambda i: (i, 0)),
            pl.BlockSpec((BLK, 1), lambda i: (i, 0)),
        ],
        out_shape=[
            jax.ShapeDtypeStruct((R, D), jnp.float32),
            jax.ShapeDtypeStruct((R, 1), jnp.float32),
        ],
    )(degp, x, W1)


def _mid_body(p_ref, y1_ref, dinv_ref, b1_ref, w2_ref, y2_ref):
    agg = p_ref[0] + p_ref[1] - y1_ref[...]
    h = jnp.maximum(agg * dinv_ref[...] + b1_ref[...], 0.0)
    y2_ref[...] = jnp.dot(h, w2_ref[...], preferred_element_type=jnp.float32,
                          precision=_prec) * dinv_ref[...]


def _mid(p, y1, dinv, b1, W2):
    return pl.pallas_call(
        _mid_body,
        grid=(NBLK,),
        in_specs=[
            pl.BlockSpec((2, BLK, D), lambda i: (0, i, 0)),
            pl.BlockSpec((BLK, D), lambda i: (i, 0)),
            pl.BlockSpec((BLK, 1), lambda i: (i, 0)),
            pl.BlockSpec((1, D), lambda i: (0, 0)),
            pl.BlockSpec((D, D), lambda i: (0, 0)),
        ],
        out_specs=pl.BlockSpec((BLK, D), lambda i: (i, 0)),
        out_shape=jax.ShapeDtypeStruct((R, D), jnp.float32),
    )(p, y1, dinv, b1, W2)


def _final_body(q_ref, y2_ref, dinv_ref, b2_ref, out_ref):
    agg = q_ref[0] + q_ref[1] - y2_ref[...]
    out_ref[...] = agg * dinv_ref[...] + b2_ref[...]


def _final(q, y2, dinv, b2):
    return pl.pallas_call(
        _final_body,
        grid=(NBLK,),
        in_specs=[
            pl.BlockSpec((2, BLK, D), lambda i: (0, i, 0)),
            pl.BlockSpec((BLK, D), lambda i: (i, 0)),
            pl.BlockSpec((BLK, 1), lambda i: (i, 0)),
            pl.BlockSpec((1, D), lambda i: (0, 0)),
        ],
        out_specs=pl.BlockSpec((BLK, D), lambda i: (i, 0)),
        out_shape=jax.ShapeDtypeStruct((N, D), jnp.float32),
    )(q, y2, dinv, b2)


def kernel(x, edge_index, W1, b1, W2, b2):
    pad = jnp.asarray(_PAD)
    src2d = jnp.reshape(jnp.concatenate([edge_index[0], pad]), (EPAD // CHUNK, CHUNK))
    dst2d = jnp.reshape(jnp.concatenate([edge_index[1], pad]), (EPAD // CHUNK, CHUNK))

    degp = _deg_kernel(dst2d).reshape(2, R, 1)
    y1, dinv = _scale_matmul(degp, x, W1)
    p = _agg_kernel(y1, src2d, dst2d)
    y2 = _mid(p, y1, dinv, b1.reshape(1, D), W2)
    q = _agg_kernel(y2, src2d, dst2d)
    return _final(q, y2, dinv, b2.reshape(1, D))


# speedup vs baseline: 1.1293x; 1.0970x over previous
import functools

import jax
import jax.numpy as jnp
import numpy as np
from jax import lax
from jax.experimental import pallas as pl
from jax.experimental.pallas import tpu as pltpu
from jax.experimental.pallas import tpu_sc as plsc

N = 10000
E = 320000
D = 128

NTILES = 32
R = 10240
RPT = R // 16
CHUNK = 128
EPW = 10240
NCHUNK = EPW // CHUNK
EPAD = NTILES * EPW
DUMMY = N

_mesh = plsc.VectorSubcoreMesh(core_axis_name="c", subcore_axis_name="s")
_prec = None

_PAD = np.asarray(N + np.arange(EPAD - E) % (R - N), np.int32)


@functools.partial(
    pl.kernel,
    out_type=jax.ShapeDtypeStruct((2, R), jnp.float32),
    mesh=_mesh,
    scratch_types=[
        pltpu.VMEM((NCHUNK // 5, CHUNK), jnp.int32),
        pltpu.VMEM((CHUNK,), jnp.float32),
        pltpu.VMEM((RPT,), jnp.float32),
        pltpu.SemaphoreType.DMA,
        pltpu.VMEM_SHARED((R,), jnp.float32),
    ],
)
def _deg_kernel(dst_hbm, out_hbm, dst_v, ones_v, zeros_v, sem, hist_sh):
    c = lax.axis_index("c")
    s = lax.axis_index("s")
    w = c * 16 + s

    def _z16(k, carry):
        zeros_v[pl.ds(k * 16, 16)] = jnp.zeros((16,), jnp.float32)
        return carry

    lax.fori_loop(0, RPT // 16, _z16, 0)

    def _o16(k, carry):
        ones_v[pl.ds(k * 16, 16)] = jnp.ones((16,), jnp.float32)
        return carry

    lax.fori_loop(0, CHUNK // 16, _o16, 0)

    pltpu.sync_copy(zeros_v, hist_sh.at[pl.ds(s * RPT, RPT)])
    plsc.subcore_barrier()

    fifth = NCHUNK // 5
    for q in range(5):
        pltpu.sync_copy(dst_hbm.at[pl.ds(w * NCHUNK + q * fifth, fifth)], dst_v)

        def _fire8(g, carry):
            for b in range(8):
                pltpu.make_async_copy(
                    ones_v, hist_sh.at[dst_v.at[g * 8 + b]], sem
                ).start(add=True)
            for b in range(8):
                pltpu.make_async_copy(
                    ones_v, hist_sh.at[dst_v.at[g * 8 + b]], sem
                ).wait()
            return carry

        lax.fori_loop(0, fifth // 8, _fire8, 0)
    plsc.subcore_barrier()
    pltpu.sync_copy(
        hist_sh.at[pl.ds(s * RPT, RPT)], out_hbm.at[c, pl.ds(s * RPT, RPT)]
    )


@functools.partial(
    pl.kernel,
    out_type=jax.ShapeDtypeStruct((2, R, D), jnp.float32),
    mesh=_mesh,
    scratch_types=[
        pltpu.VMEM((NCHUNK // 2, CHUNK), jnp.int32),
        pltpu.VMEM((NCHUNK // 2, CHUNK), jnp.int32),
        pltpu.VMEM((2, CHUNK, D), jnp.float32),
        pltpu.SemaphoreType.DMA,
        pltpu.SemaphoreType.DMA,
        pltpu.SemaphoreType.DMA,
        pltpu.SemaphoreType.DMA,
        pltpu.VMEM_SHARED((R, D), jnp.float32),
    ],
)
def _agg_kernel(y_hbm, src_hbm, dst_hbm, out_hbm, src_v, dst_v, rows_v,
                gsem0, gsem1, ssem0, ssem1, acc_sh):
    c = lax.axis_index("c")
    s = lax.axis_index("s")
    w = c * 16 + s
    pltpu.sync_copy(y_hbm.at[pl.ds(s * RPT, RPT)], acc_sh.at[pl.ds(s * RPT, RPT)])
    plsc.subcore_barrier()

    gsems = (gsem0, gsem1)
    ssems = (ssem0, ssem1)
    half = NCHUNK // 2
    for h in range(2):
        base = w * NCHUNK + h * half
        pltpu.sync_copy(src_hbm.at[pl.ds(base, half)], src_v)
        pltpu.sync_copy(dst_hbm.at[pl.ds(base, half)], dst_v)
        pltpu.make_async_copy(y_hbm.at[src_v.at[0]], rows_v.at[0], gsem0).start()

        def _body(t, carry):
            for b in range(2):
                j = t * 2 + b
                nxt = j + 1

                @pl.when(j >= 1)
                def _():
                    pltpu.make_async_copy(
                        rows_v.at[1 - b], acc_sh.at[dst_v.at[0]], ssems[1 - b]
                    ).wait()

                @pl.when(nxt < half)
                def _():
                    pltpu.make_async_copy(
                        y_hbm.at[src_v.at[nxt]], rows_v.at[1 - b], gsems[1 - b]
                    ).start()

                pltpu.make_async_copy(
                    y_hbm.at[src_v.at[j]], rows_v.at[b], gsems[b]
                ).wait()
                pltpu.make_async_copy(
                    rows_v.at[b], acc_sh.at[dst_v.at[j]], ssems[b]
                ).start(add=True)
            return carry

        lax.fori_loop(0, half // 2, _body, 0)
        pltpu.make_async_copy(rows_v.at[1], acc_sh.at[dst_v.at[0]], ssem1).wait()
    plsc.subcore_barrier()
    pltpu.sync_copy(
        acc_sh.at[pl.ds(s * RPT, RPT)], out_hbm.at[c, pl.ds(s * RPT, RPT)]
    )


BLK = 2000
NBLK = N // BLK


def _scale_mm_body(degp_ref, x_ref, w_ref, y_ref, dinv_ref):
    deg = degp_ref[0] + degp_ref[1] + 1.0
    dinv = lax.rsqrt(deg)
    xw = jnp.dot(x_ref[...], w_ref[...], preferred_element_type=jnp.float32,
                 precision=_prec)
    y_ref[...] = xw * dinv
    dinv_ref[...] = dinv


def _scale_matmul(degp, x, W1):
    return pl.pallas_call(
        _scale_mm_body,
        grid=(NBLK,),
        in_specs=[
            pl.BlockSpec((2, BLK, 1), lambda i: (0, i, 0)),
            pl.BlockSpec((BLK, D), lambda i: (i, 0)),
            pl.BlockSpec((D, D), lambda i: (0, 0)),
        ],
        out_specs=[
            pl.BlockSpec((BLK, D), lambda i: (i, 0)),
            pl.BlockSpec((BLK, 1), lambda i: (i, 0)),
        ],
        out_shape=[
            jax.ShapeDtypeStruct((R, D), jnp.float32),
            jax.ShapeDtypeStruct((R, 1), jnp.float32),
        ],
    )(degp, x, W1)


def _mid_body(p_ref, y1_ref, dinv_ref, b1_ref, w2_ref, y2_ref):
    agg = p_ref[0] + p_ref[1] - y1_ref[...]
    h = jnp.maximum(agg * dinv_ref[...] + b1_ref[...], 0.0)
    y2_ref[...] = jnp.dot(h, w2_ref[...], preferred_element_type=jnp.float32,
                          precision=_prec) * dinv_ref[...]


def _mid(p, y1, dinv, b1, W2):
    return pl.pallas_call(
        _mid_body,
        grid=(NBLK,),
        in_specs=[
            pl.BlockSpec((2, BLK, D), lambda i: (0, i, 0)),
            pl.BlockSpec((BLK, D), lambda i: (i, 0)),
            pl.BlockSpec((BLK, 1), lambda i: (i, 0)),
            pl.BlockSpec((1, D), lambda i: (0, 0)),
            pl.BlockSpec((D, D), lambda i: (0, 0)),
        ],
        out_specs=pl.BlockSpec((BLK, D), lambda i: (i, 0)),
        out_shape=jax.ShapeDtypeStruct((R, D), jnp.float32),
    )(p, y1, dinv, b1, W2)


def _final_body(q_ref, y2_ref, dinv_ref, b2_ref, out_ref):
    agg = q_ref[0] + q_ref[1] - y2_ref[...]
    out_ref[...] = agg * dinv_ref[...] + b2_ref[...]


def _final(q, y2, dinv, b2):
    return pl.pallas_call(
        _final_body,
        grid=(NBLK,),
        in_specs=[
            pl.BlockSpec((2, BLK, D), lambda i: (0, i, 0)),
            pl.BlockSpec((BLK, D), lambda i: (i, 0)),
            pl.BlockSpec((BLK, 1), lambda i: (i, 0)),
            pl.BlockSpec((1, D), lambda i: (0, 0)),
        ],
        out_specs=pl.BlockSpec((BLK, D), lambda i: (i, 0)),
        out_shape=jax.ShapeDtypeStruct((N, D), jnp.float32),
    )(q, y2, dinv, b2)


def kernel(x, edge_index, W1, b1, W2, b2):
    pad2d = jnp.asarray(_PAD.reshape(-1, CHUNK))
    src2d = jnp.concatenate([edge_index[0].reshape(E // CHUNK, CHUNK), pad2d])
    dst2d = jnp.concatenate([edge_index[1].reshape(E // CHUNK, CHUNK), pad2d])

    degp = _deg_kernel(dst2d).reshape(2, R, 1)
    y1, dinv = _scale_matmul(degp, x, W1)
    p = _agg_kernel(y1, src2d, dst2d)
    y2 = _mid(p, y1, dinv, b1.reshape(1, D), W2)
    q = _agg_kernel(y2, src2d, dst2d)
    return _final(q, y2, dinv, b2.reshape(1, D))


# speedup vs baseline: 1.1637x; 1.0305x over previous
import functools

import jax
import jax.numpy as jnp
import numpy as np
from jax import lax
from jax.experimental import pallas as pl
from jax.experimental.pallas import tpu as pltpu
from jax.experimental.pallas import tpu_sc as plsc

N = 10000
E = 320000
D = 128

NTILES = 32
R = 10240
RPT = R // 16
CHUNK = 128
EPW = 10240
NCHUNK = EPW // CHUNK
EPAD = NTILES * EPW
DUMMY = N

_mesh = plsc.VectorSubcoreMesh(core_axis_name="c", subcore_axis_name="s")
_prec = None

_PAD = np.asarray(N + np.arange(EPAD - E) % (R - N), np.int32)


@functools.partial(
    pl.kernel,
    out_type=jax.ShapeDtypeStruct((2, R), jnp.float32),
    mesh=_mesh,
    scratch_types=[
        pltpu.VMEM((NCHUNK // 5, CHUNK), jnp.int32),
        pltpu.VMEM((CHUNK,), jnp.float32),
        pltpu.VMEM((RPT,), jnp.float32),
        pltpu.SemaphoreType.DMA,
        pltpu.VMEM_SHARED((R,), jnp.float32),
    ],
)
def _deg_kernel(dst_hbm, out_hbm, dst_v, ones_v, zeros_v, sem, hist_sh):
    c = lax.axis_index("c")
    s = lax.axis_index("s")
    w = c * 16 + s

    def _z16(k, carry):
        zeros_v[pl.ds(k * 16, 16)] = jnp.zeros((16,), jnp.float32)
        return carry

    lax.fori_loop(0, RPT // 16, _z16, 0)

    def _o16(k, carry):
        ones_v[pl.ds(k * 16, 16)] = jnp.ones((16,), jnp.float32)
        return carry

    lax.fori_loop(0, CHUNK // 16, _o16, 0)

    pltpu.sync_copy(zeros_v, hist_sh.at[pl.ds(s * RPT, RPT)])
    plsc.subcore_barrier()

    fifth = NCHUNK // 5
    for q in range(5):
        pltpu.sync_copy(dst_hbm.at[pl.ds(w * NCHUNK + q * fifth, fifth)], dst_v)

        def _fire8(g, carry):
            for b in range(8):
                pltpu.make_async_copy(
                    ones_v, hist_sh.at[dst_v.at[g * 8 + b]], sem
                ).start(add=True)
            for b in range(8):
                pltpu.make_async_copy(
                    ones_v, hist_sh.at[dst_v.at[g * 8 + b]], sem
                ).wait()
            return carry

        lax.fori_loop(0, fifth // 8, _fire8, 0)
    plsc.subcore_barrier()
    pltpu.sync_copy(
        hist_sh.at[pl.ds(s * RPT, RPT)], out_hbm.at[c, pl.ds(s * RPT, RPT)]
    )


@functools.partial(
    pl.kernel,
    out_type=jax.ShapeDtypeStruct((2, R, D), jnp.float32),
    mesh=_mesh,
    scratch_types=[
        pltpu.VMEM((NCHUNK // 2, CHUNK), jnp.int32),
        pltpu.VMEM((NCHUNK // 2, CHUNK), jnp.int32),
        pltpu.VMEM((2, CHUNK, D), jnp.float32),
        pltpu.SemaphoreType.DMA,
        pltpu.SemaphoreType.DMA,
        pltpu.SemaphoreType.DMA,
        pltpu.SemaphoreType.DMA,
        pltpu.VMEM_SHARED((R, D), jnp.float32),
    ],
)
def _agg_kernel(y_hbm, src_hbm, dst_hbm, out_hbm, src_v, dst_v, rows_v,
                gsem0, gsem1, ssem0, ssem1, acc_sh):
    c = lax.axis_index("c")
    s = lax.axis_index("s")
    w = c * 16 + s
    pltpu.sync_copy(y_hbm.at[pl.ds(s * RPT, RPT)], acc_sh.at[pl.ds(s * RPT, RPT)])
    plsc.subcore_barrier()

    gsems = (gsem0, gsem1)
    ssems = (ssem0, ssem1)
    half = NCHUNK // 2
    for h in range(2):
        base = w * NCHUNK + h * half
        pltpu.sync_copy(src_hbm.at[pl.ds(base, half)], src_v)
        pltpu.sync_copy(dst_hbm.at[pl.ds(base, half)], dst_v)
        pltpu.make_async_copy(y_hbm.at[src_v.at[0]], rows_v.at[0], gsem0).start()

        def _body(t, carry):
            for b in range(2):
                j = t * 2 + b
                nxt = j + 1

                @pl.when(j >= 1)
                def _():
                    pltpu.make_async_copy(
                        rows_v.at[1 - b], acc_sh.at[dst_v.at[0]], ssems[1 - b]
                    ).wait()

                @pl.when(nxt < half)
                def _():
                    pltpu.make_async_copy(
                        y_hbm.at[src_v.at[nxt]], rows_v.at[1 - b], gsems[1 - b]
                    ).start()

                pltpu.make_async_copy(
                    y_hbm.at[src_v.at[j]], rows_v.at[b], gsems[b]
                ).wait()
                pltpu.make_async_copy(
                    rows_v.at[b], acc_sh.at[dst_v.at[j]], ssems[b]
                ).start(add=True)
            return carry

        lax.fori_loop(0, half // 2, _body, 0)
        pltpu.make_async_copy(rows_v.at[1], acc_sh.at[dst_v.at[0]], ssem1).wait()
    plsc.subcore_barrier()
    pltpu.sync_copy(
        acc_sh.at[pl.ds(s * RPT, RPT)], out_hbm.at[c, pl.ds(s * RPT, RPT)]
    )


BLK = 2000
NBLK = N // BLK


TBLK = 512


def _dinv_body(degp_ref, dinv_ref):
    deg = (degp_ref[0] + degp_ref[1] + 1.0).reshape(1, TBLK)
    dv = lax.rsqrt(deg)
    ri = lax.broadcasted_iota(jnp.int32, (128, 128), 0)
    ci = lax.broadcasted_iota(jnp.int32, (128, 128), 1)
    eye = ri == ci
    for k in range(TBLK // 128):
        seg = jnp.broadcast_to(dv[:, k * 128:(k + 1) * 128], (128, 128))
        col = jnp.sum(jnp.where(eye, seg, 0.0), axis=1, keepdims=True)
        dinv_ref[pl.ds(k * 128, 128), :] = col


def _dinv_kernel(degp):
    return pl.pallas_call(
        _dinv_body,
        grid=(R // TBLK,),
        in_specs=[pl.BlockSpec((2, TBLK), lambda i: (0, i))],
        out_specs=pl.BlockSpec((TBLK, 1), lambda i: (i, 0)),
        out_shape=jax.ShapeDtypeStruct((R, 1), jnp.float32),
    )(degp)


def _scale_mm_body(dinv_ref, x_ref, w_ref, y_ref):
    xw = jnp.dot(x_ref[...], w_ref[...], preferred_element_type=jnp.float32,
                 precision=_prec)
    y_ref[...] = xw * dinv_ref[...]


def _scale_matmul(dinv, x, W1):
    return pl.pallas_call(
        _scale_mm_body,
        grid=(NBLK,),
        in_specs=[
            pl.BlockSpec((BLK, 1), lambda i: (i, 0)),
            pl.BlockSpec((BLK, D), lambda i: (i, 0)),
            pl.BlockSpec((D, D), lambda i: (0, 0)),
        ],
        out_specs=pl.BlockSpec((BLK, D), lambda i: (i, 0)),
        out_shape=jax.ShapeDtypeStruct((R, D), jnp.float32),
    )(dinv, x, W1)


def _mid_body(p_ref, y1_ref, dinv_ref, b1_ref, w2_ref, y2_ref):
    agg = p_ref[0] + p_ref[1] - y1_ref[...]
    h = jnp.maximum(agg * dinv_ref[...] + b1_ref[...], 0.0)
    y2_ref[...] = jnp.dot(h, w2_ref[...], preferred_element_type=jnp.float32,
                          precision=_prec) * dinv_ref[...]


def _mid(p, y1, dinv, b1, W2):
    return pl.pallas_call(
        _mid_body,
        grid=(NBLK,),
        in_specs=[
            pl.BlockSpec((2, BLK, D), lambda i: (0, i, 0)),
            pl.BlockSpec((BLK, D), lambda i: (i, 0)),
            pl.BlockSpec((BLK, 1), lambda i: (i, 0)),
            pl.BlockSpec((1, D), lambda i: (0, 0)),
            pl.BlockSpec((D, D), lambda i: (0, 0)),
        ],
        out_specs=pl.BlockSpec((BLK, D), lambda i: (i, 0)),
        out_shape=jax.ShapeDtypeStruct((R, D), jnp.float32),
    )(p, y1, dinv, b1, W2)


def _final_body(q_ref, y2_ref, dinv_ref, b2_ref, out_ref):
    agg = q_ref[0] + q_ref[1] - y2_ref[...]
    out_ref[...] = agg * dinv_ref[...] + b2_ref[...]


def _final(q, y2, dinv, b2):
    return pl.pallas_call(
        _final_body,
        grid=(NBLK,),
        in_specs=[
            pl.BlockSpec((2, BLK, D), lambda i: (0, i, 0)),
            pl.BlockSpec((BLK, D), lambda i: (i, 0)),
            pl.BlockSpec((BLK, 1), lambda i: (i, 0)),
            pl.BlockSpec((1, D), lambda i: (0, 0)),
        ],
        out_specs=pl.BlockSpec((BLK, D), lambda i: (i, 0)),
        out_shape=jax.ShapeDtypeStruct((N, D), jnp.float32),
    )(q, y2, dinv, b2)


def kernel(x, edge_index, W1, b1, W2, b2):
    pad2 = jnp.asarray(np.stack([_PAD, _PAD]))
    e3d = jnp.concatenate([edge_index, pad2], axis=1).reshape(2, EPAD // CHUNK, CHUNK)
    src2d = e3d[0]
    dst2d = e3d[1]

    degp = _deg_kernel(dst2d)
    dinv = _dinv_kernel(degp)
    y1 = _scale_matmul(dinv, x, W1)
    p = _agg_kernel(y1, src2d, dst2d)
    y2 = _mid(p, y1, dinv, b1.reshape(1, D), W2)
    q = _agg_kernel(y2, src2d, dst2d)
    return _final(q, y2, dinv, b2.reshape(1, D))


# speedup vs baseline: 1.1648x; 1.0009x over previous
import functools

import jax
import jax.numpy as jnp
import numpy as np
from jax import lax
from jax.experimental import pallas as pl
from jax.experimental.pallas import tpu as pltpu
from jax.experimental.pallas import tpu_sc as plsc

N = 10000
E = 320000
D = 128

NTILES = 32
R = 10240
RPT = R // 16
CHUNK = 128
EPW = 10240
NCHUNK = EPW // CHUNK
EPAD = NTILES * EPW

_mesh = plsc.VectorSubcoreMesh(core_axis_name="c", subcore_axis_name="s")
_prec = None

_PAD = np.asarray(N + np.arange(EPAD - E) % (R - N), np.int32)


@functools.partial(
    pl.kernel,
    out_type=jax.ShapeDtypeStruct((2, R), jnp.float32),
    mesh=_mesh,
    scratch_types=[
        pltpu.VMEM((NCHUNK // 5, CHUNK), jnp.int32),
        pltpu.VMEM((CHUNK,), jnp.float32),
        pltpu.VMEM((RPT,), jnp.float32),
        pltpu.SemaphoreType.DMA,
        pltpu.VMEM_SHARED((R,), jnp.float32),
    ],
)
def _deg_kernel(dst_hbm, out_hbm, dst_v, ones_v, zeros_v, sem, hist_sh):
    c = lax.axis_index("c")
    s = lax.axis_index("s")
    w = c * 16 + s

    def _z16(k, carry):
        zeros_v[pl.ds(k * 16, 16)] = jnp.zeros((16,), jnp.float32)
        return carry

    lax.fori_loop(0, RPT // 16, _z16, 0)

    def _o16(k, carry):
        ones_v[pl.ds(k * 16, 16)] = jnp.ones((16,), jnp.float32)
        return carry

    lax.fori_loop(0, CHUNK // 16, _o16, 0)

    pltpu.sync_copy(zeros_v, hist_sh.at[pl.ds(s * RPT, RPT)])
    plsc.subcore_barrier()

    fifth = NCHUNK // 5
    for q in range(5):
        pltpu.sync_copy(dst_hbm.at[pl.ds(w * NCHUNK + q * fifth, fifth)], dst_v)

        def _fire8(g, carry):
            for b in range(8):
                pltpu.make_async_copy(
                    ones_v, hist_sh.at[dst_v.at[g * 8 + b]], sem
                ).start(add=True)
            for b in range(8):
                pltpu.make_async_copy(
                    ones_v, hist_sh.at[dst_v.at[g * 8 + b]], sem
                ).wait()
            return carry

        lax.fori_loop(0, fifth // 8, _fire8, 0)
    plsc.subcore_barrier()
    pltpu.sync_copy(
        hist_sh.at[pl.ds(s * RPT, RPT)], out_hbm.at[c, pl.ds(s * RPT, RPT)]
    )


@functools.partial(
    pl.kernel,
    out_type=jax.ShapeDtypeStruct((2, R, D), jnp.float32),
    mesh=_mesh,
    scratch_types=[
        pltpu.VMEM((NCHUNK // 2, CHUNK), jnp.int32),
        pltpu.VMEM((NCHUNK // 2, CHUNK), jnp.int32),
        pltpu.VMEM((2, CHUNK, D), jnp.float32),
        pltpu.SemaphoreType.DMA,
        pltpu.SemaphoreType.DMA,
        pltpu.SemaphoreType.DMA,
        pltpu.SemaphoreType.DMA,
        pltpu.VMEM_SHARED((R, D), jnp.float32),
    ],
)
def _agg_kernel(y_hbm, src_hbm, dst_hbm, out_hbm, src_v, dst_v, rows_v,
                gsem0, gsem1, ssem0, ssem1, acc_sh):
    c = lax.axis_index("c")
    s = lax.axis_index("s")
    w = c * 16 + s
    pltpu.sync_copy(y_hbm.at[pl.ds(s * RPT, RPT)], acc_sh.at[pl.ds(s * RPT, RPT)])
    plsc.subcore_barrier()

    gsems = (gsem0, gsem1)
    ssems = (ssem0, ssem1)
    half = NCHUNK // 2
    for h in range(2):
        base = w * NCHUNK + h * half
        pltpu.sync_copy(src_hbm.at[pl.ds(base, half)], src_v)
        pltpu.sync_copy(dst_hbm.at[pl.ds(base, half)], dst_v)
        pltpu.make_async_copy(y_hbm.at[src_v.at[0]], rows_v.at[0], gsem0).start()

        def _body(t, carry):
            for b in range(2):
                j = t * 2 + b
                nxt = j + 1

                @pl.when(j >= 1)
                def _():
                    pltpu.make_async_copy(
                        rows_v.at[1 - b], acc_sh.at[dst_v.at[0]], ssems[1 - b]
                    ).wait()

                @pl.when(nxt < half)
                def _():
                    pltpu.make_async_copy(
                        y_hbm.at[src_v.at[nxt]], rows_v.at[1 - b], gsems[1 - b]
                    ).start()

                pltpu.make_async_copy(
                    y_hbm.at[src_v.at[j]], rows_v.at[b], gsems[b]
                ).wait()
                pltpu.make_async_copy(
                    rows_v.at[b], acc_sh.at[dst_v.at[j]], ssems[b]
                ).start(add=True)
            return carry

        lax.fori_loop(0, half // 2, _body, 0)
        pltpu.make_async_copy(rows_v.at[1], acc_sh.at[dst_v.at[0]], ssem1).wait()
    plsc.subcore_barrier()
    pltpu.sync_copy(
        acc_sh.at[pl.ds(s * RPT, RPT)], out_hbm.at[c, pl.ds(s * RPT, RPT)]
    )


BLK = 2000
NBLK = N // BLK


TBLK = 512


def _dinv_body(degp_ref, dinv_ref):
    deg = (degp_ref[0] + degp_ref[1] + 1.0).reshape(1, TBLK)
    dv = lax.rsqrt(deg)
    ri = lax.broadcasted_iota(jnp.int32, (128, 128), 0)
    ci = lax.broadcasted_iota(jnp.int32, (128, 128), 1)
    eye = ri == ci
    for k in range(TBLK // 128):
        seg = jnp.broadcast_to(dv[:, k * 128:(k + 1) * 128], (128, 128))
        col = jnp.sum(jnp.where(eye, seg, 0.0), axis=1, keepdims=True)
        dinv_ref[pl.ds(k * 128, 128), :] = col


def _dinv_kernel(degp):
    return pl.pallas_call(
        _dinv_body,
        grid=(R // TBLK,),
        in_specs=[pl.BlockSpec((2, TBLK), lambda i: (0, i))],
        out_specs=pl.BlockSpec((TBLK, 1), lambda i: (i, 0)),
        out_shape=jax.ShapeDtypeStruct((R, 1), jnp.float32),
    )(degp)


def _scale_mm_body(dinv_ref, x_ref, w_ref, y_ref):
    xw = jnp.dot(x_ref[...], w_ref[...], preferred_element_type=jnp.float32,
                 precision=_prec)
    y_ref[...] = xw * dinv_ref[...]


def _scale_matmul(dinv, x, W1):
    return pl.pallas_call(
        _scale_mm_body,
        grid=(NBLK,),
        in_specs=[
            pl.BlockSpec((BLK, 1), lambda i: (i, 0)),
            pl.BlockSpec((BLK, D), lambda i: (i, 0)),
            pl.BlockSpec((D, D), lambda i: (0, 0)),
        ],
        out_specs=pl.BlockSpec((BLK, D), lambda i: (i, 0)),
        out_shape=jax.ShapeDtypeStruct((R, D), jnp.float32),
    )(dinv, x, W1)


def _mid_body(p_ref, y1_ref, dinv_ref, b1_ref, w2_ref, y2_ref):
    agg = p_ref[0] + p_ref[1] - y1_ref[...]
    h = jnp.maximum(agg * dinv_ref[...] + b1_ref[...], 0.0)
    y2_ref[...] = jnp.dot(h, w2_ref[...], preferred_element_type=jnp.float32,
                          precision=_prec) * dinv_ref[...]


def _mid(p, y1, dinv, b1, W2):
    return pl.pallas_call(
        _mid_body,
        grid=(NBLK,),
        in_specs=[
            pl.BlockSpec((2, BLK, D), lambda i: (0, i, 0)),
            pl.BlockSpec((BLK, D), lambda i: (i, 0)),
            pl.BlockSpec((BLK, 1), lambda i: (i, 0)),
            pl.BlockSpec((1, D), lambda i: (0, 0)),
            pl.BlockSpec((D, D), lambda i: (0, 0)),
        ],
        out_specs=pl.BlockSpec((BLK, D), lambda i: (i, 0)),
        out_shape=jax.ShapeDtypeStruct((R, D), jnp.float32),
    )(p, y1, dinv, b1, W2)


def _final_body(q_ref, y2_ref, dinv_ref, b2_ref, out_ref):
    agg = q_ref[0] + q_ref[1] - y2_ref[...]
    out_ref[...] = agg * dinv_ref[...] + b2_ref[...]


def _final(q, y2, dinv, b2):
    return pl.pallas_call(
        _final_body,
        grid=(NBLK,),
        in_specs=[
            pl.BlockSpec((2, BLK, D), lambda i: (0, i, 0)),
            pl.BlockSpec((BLK, D), lambda i: (i, 0)),
            pl.BlockSpec((BLK, 1), lambda i: (i, 0)),
            pl.BlockSpec((1, D), lambda i: (0, 0)),
        ],
        out_specs=pl.BlockSpec((BLK, D), lambda i: (i, 0)),
        out_shape=jax.ShapeDtypeStruct((N, D), jnp.float32),
    )(q, y2, dinv, b2)


def kernel(x, edge_index, W1, b1, W2, b2):
    pad2 = jnp.asarray(np.stack([_PAD, _PAD]))
    e3d = jnp.concatenate([edge_index, pad2], axis=1).reshape(2, EPAD // CHUNK, CHUNK)
    src2d = e3d[0]
    dst2d = e3d[1]

    degp = _deg_kernel(dst2d)
    dinv = _dinv_kernel(degp)
    y1 = _scale_matmul(dinv, x, W1)
    p = _agg_kernel(y1, src2d, dst2d)
    y2 = _mid(p, y1, dinv, b1.reshape(1, D), W2)
    q = _agg_kernel(y2, src2d, dst2d)
    return _final(q, y2, dinv, b2.reshape(1, D))
